# baseline (speedup 1.0000x reference)
import functools

import jax
import jax.numpy as jnp
from jax import lax
from jax.experimental import pallas as pl
from jax.experimental.pallas import tpu as pltpu
from jax.experimental.pallas import tpu_sc as plsc

N_FIELDS = 26
VOCAB = 100000
EMB_DIM = 32

NC = 2
NS = 16
NW = NC * NS

BCHUNK = 4096


@functools.lru_cache(maxsize=None)
def _make_lookup(batch: int):
    n_rows = N_FIELDS * EMB_DIM
    units = n_rows // NW
    assert units * NW == n_rows and batch % (2 * BCHUNK) == 0
    n_bchunks = batch // BCHUNK

    mesh = plsc.VectorSubcoreMesh(
        core_axis_name="c", subcore_axis_name="s", num_cores=NC, num_subcores=NS
    )

    def body(tt, catT, out, vec, idxb0, idxb1, outb0, outb1, vsem, isem, osem):
        wid = lax.axis_index("s") * NC + lax.axis_index("c")
        idxbs = (idxb0, idxb1)
        outbs = (outb0, outb1)

        def start_idx(f, c, b):
            pltpu.async_copy(
                catT.at[f, pl.ds(c * BCHUNK, BCHUNK)], idxbs[b], isem.at[b]
            )

        def wait_idx(b):
            pltpu.make_async_copy(
                catT.at[0, pl.ds(0, BCHUNK)], idxbs[b], isem.at[b]
            ).wait()

        def start_out(u, c, b):
            pltpu.async_copy(
                outbs[b], out.at[u, pl.ds(c * BCHUNK, BCHUNK)], osem.at[b]
            )

        def wait_out(b):
            pltpu.make_async_copy(
                outbs[b], out.at[0, pl.ds(0, BCHUNK)], osem.at[b]
            ).wait()

        def gather_chunk(b):
            ib = idxbs[b]
            ob = outbs[b]

            @pl.loop(0, BCHUNK // 16, unroll=16)
            def _(k):
                iv = ib[pl.ds(k * 16, 16)]
                ob[pl.ds(k * 16, 16)] = plsc.load_gather(vec, [iv])

        @pl.loop(0, units)
        def _(t):
            u = wid * units + t
            f = u // EMB_DIM
            e = lax.rem(u, EMB_DIM)
            pltpu.async_copy(tt.at[f, e], vec, vsem)
            start_idx(f, 0, 0)
            start_idx(f, 1, 1)
            pltpu.make_async_copy(tt.at[0, 0], vec, vsem).wait()

            @pl.loop(0, n_bchunks, step=2)
            def _(c0):
                for b in range(2):
                    c = c0 + b
                    wait_idx(b)

                    @pl.when(t * n_bchunks + c > 1)
                    def _():
                        wait_out(b)

                    gather_chunk(b)
                    start_out(u, c, b)
                    nc = c + 2

                    @pl.when(nc < n_bchunks)
                    def _():
                        start_idx(f, nc, b)

        for b in range(2):
            wait_out(b)

    return pl.kernel(
        body,
        out_type=jax.ShapeDtypeStruct((n_rows, batch), jnp.float32),
        mesh=mesh,
        compiler_params=pltpu.CompilerParams(
            use_tc_tiling_on_sc=True, needs_layout_passes=False
        ),
        scratch_types=[
            pltpu.VMEM((VOCAB,), jnp.float32),
            pltpu.VMEM((BCHUNK,), jnp.int32),
            pltpu.VMEM((BCHUNK,), jnp.int32),
            pltpu.VMEM((BCHUNK,), jnp.float32),
            pltpu.VMEM((BCHUNK,), jnp.float32),
            pltpu.SemaphoreType.DMA,
            pltpu.SemaphoreType.DMA((2,)),
            pltpu.SemaphoreType.DMA((2,)),
        ],
    )


def kernel(cat_features, tables):
    batch = cat_features.shape[0]
    cat = cat_features.astype(jnp.int32)
    tt = jnp.transpose(tables, (0, 2, 1))
    catT = jnp.transpose(cat, (1, 0))
    out = _make_lookup(batch)(tt, catT)
    return jnp.transpose(out, (1, 0))


# speedup vs baseline: 1.6684x; 1.6684x over previous
import functools

import jax
import jax.numpy as jnp
from jax import lax
from jax.experimental import pallas as pl
from jax.experimental.pallas import tpu as pltpu
from jax.experimental.pallas import tpu_sc as plsc

N_FIELDS = 26
VOCAB = 100000
EMB_DIM = 32

NC = 2
NS = 16
NW = NC * NS

BCHUNK = 2048


@functools.lru_cache(maxsize=None)
def _make_lookup(batch: int):
    n_rows = N_FIELDS * EMB_DIM
    units = n_rows // NW
    assert units * NW == n_rows and batch % (2 * BCHUNK) == 0
    n_bchunks = batch // BCHUNK

    mesh = plsc.VectorSubcoreMesh(
        core_axis_name="c", subcore_axis_name="s", num_cores=NC, num_subcores=NS
    )

    def body(tt, catT, out, vec, idxb0, idxb1, outb0, outb1, vsem, isem, osem):
        wid = lax.axis_index("s") * NC + lax.axis_index("c")
        idxbs = (idxb0, idxb1)
        outbs = (outb0, outb1)

        def start_idx(f, c, b):
            pltpu.async_copy(
                catT.at[f, pl.ds(c * BCHUNK, BCHUNK)], idxbs[b], isem.at[b]
            )

        def wait_idx(b):
            pltpu.make_async_copy(
                catT.at[0, pl.ds(0, BCHUNK)], idxbs[b], isem.at[b]
            ).wait()

        def start_out(u, c, b):
            pltpu.async_copy(
                outbs[b], out.at[u, pl.ds(c * BCHUNK, BCHUNK)], osem.at[b]
            )

        def wait_out(b):
            pltpu.make_async_copy(
                outbs[b], out.at[0, pl.ds(0, BCHUNK)], osem.at[b]
            ).wait()

        def gather_chunk(b):
            ib = idxbs[b]
            ob = outbs[b]

            @plsc.parallel_loop(0, BCHUNK // 16, unroll=8)
            def _(k):
                iv = ib[pl.ds(k * 16, 16)]
                ob[pl.ds(k * 16, 16)] = plsc.load_gather(vec, [iv])

        @pl.loop(0, units)
        def _(t):
            u = wid * units + t
            f = u // EMB_DIM
            e = lax.rem(u, EMB_DIM)
            pltpu.async_copy(tt.at[f, e], vec, vsem)
            start_idx(f, 0, 0)
            start_idx(f, 1, 1)
            pltpu.make_async_copy(tt.at[0, 0], vec, vsem).wait()

            @pl.loop(0, n_bchunks, step=2)
            def _(c0):
                for b in range(2):
                    c = c0 + b
                    wait_idx(b)

                    @pl.when(t * n_bchunks + c > 1)
                    def _():
                        wait_out(b)

                    gather_chunk(b)
                    start_out(u, c, b)
                    nc = c + 2

                    @pl.when(nc < n_bchunks)
                    def _():
                        start_idx(f, nc, b)

        for b in range(2):
            wait_out(b)

    return pl.kernel(
        body,
        out_type=jax.ShapeDtypeStruct((n_rows, batch), jnp.float32),
        mesh=mesh,
        compiler_params=pltpu.CompilerParams(
            use_tc_tiling_on_sc=True, needs_layout_passes=False
        ),
        scratch_types=[
            pltpu.VMEM((VOCAB,), jnp.float32),
            pltpu.VMEM((BCHUNK,), jnp.int32),
            pltpu.VMEM((BCHUNK,), jnp.int32),
            pltpu.VMEM((BCHUNK,), jnp.float32),
            pltpu.VMEM((BCHUNK,), jnp.float32),
            pltpu.SemaphoreType.DMA,
            pltpu.SemaphoreType.DMA((2,)),
            pltpu.SemaphoreType.DMA((2,)),
        ],
    )


def kernel(cat_features, tables):
    batch = cat_features.shape[0]
    cat = cat_features.astype(jnp.int32)
    tt = jnp.transpose(tables, (0, 2, 1))
    catT = jnp.transpose(cat, (1, 0))
    out = _make_lookup(batch)(tt, catT)
    return jnp.transpose(out, (1, 0))


# speedup vs baseline: 2.0785x; 1.2458x over previous
import functools

import jax
import jax.numpy as jnp
from jax import lax
from jax.experimental import pallas as pl
from jax.experimental.pallas import tpu as pltpu
from jax.experimental.pallas import tpu_sc as plsc

N_FIELDS = 26
VOCAB = 100000
EMB_DIM = 32

NC = 2
NS = 16
NW = NC * NS

BCHUNK = 2048


@functools.lru_cache(maxsize=None)
def _make_lookup(batch: int):
    n_rows = N_FIELDS * EMB_DIM
    units = n_rows // NW
    assert units * NW == n_rows and batch % (2 * BCHUNK) == 0
    n_bchunks = batch // BCHUNK

    mesh = plsc.VectorSubcoreMesh(
        core_axis_name="c", subcore_axis_name="s", num_cores=NC, num_subcores=NS
    )

    def body(tt, catT, out, vec, idxrow, outb0, outb1, vsem, isem, osem):
        wid = lax.axis_index("s") * NC + lax.axis_index("c")
        outbs = (outb0, outb1)

        def start_out(u, c, b):
            pltpu.async_copy(
                outbs[b], out.at[u, pl.ds(c * BCHUNK, BCHUNK)], osem.at[b]
            )

        def wait_out(b):
            pltpu.make_async_copy(
                outbs[b], out.at[0, pl.ds(0, BCHUNK)], osem.at[b]
            ).wait()

        def gather_chunk(c, b):
            ob = outbs[b]
            base = c * BCHUNK

            @plsc.parallel_loop(0, BCHUNK // 16, unroll=8)
            def _(k):
                iv = idxrow[pl.ds(base + k * 16, 16)]
                ob[pl.ds(k * 16, 16)] = plsc.load_gather(vec, [iv])

        @pl.loop(0, units)
        def _(t):
            u = wid * units + t
            f = u // EMB_DIM
            e = lax.rem(u, EMB_DIM)
            pltpu.async_copy(tt.at[f, e], vec, vsem)

            @pl.when((t == 0) | (e == 0))
            def _():
                pltpu.async_copy(catT.at[f], idxrow, isem)
                pltpu.make_async_copy(catT.at[0], idxrow, isem).wait()

            pltpu.make_async_copy(tt.at[0, 0], vec, vsem).wait()

            @pl.loop(0, n_bchunks, step=2)
            def _(c0):
                for b in range(2):
                    c = c0 + b

                    @pl.when(t * n_bchunks + c > 1)
                    def _():
                        wait_out(b)

                    gather_chunk(c, b)
                    start_out(u, c, b)

        for b in range(2):
            wait_out(b)

    return pl.kernel(
        body,
        out_type=jax.ShapeDtypeStruct((n_rows, batch), jnp.float32),
        mesh=mesh,
        compiler_params=pltpu.CompilerParams(
            use_tc_tiling_on_sc=True, needs_layout_passes=False
        ),
        scratch_types=[
            pltpu.VMEM((VOCAB,), jnp.float32),
            pltpu.VMEM((batch,), jnp.int32),
            pltpu.VMEM((BCHUNK,), jnp.float32),
            pltpu.VMEM((BCHUNK,), jnp.float32),
            pltpu.SemaphoreType.DMA,
            pltpu.SemaphoreType.DMA,
            pltpu.SemaphoreType.DMA((2,)),
        ],
    )


def kernel(cat_features, tables):
    batch = cat_features.shape[0]
    cat = cat_features.astype(jnp.int32)
    tt = jnp.transpose(tables, (0, 2, 1))
    catT = jnp.transpose(cat, (1, 0))
    out = _make_lookup(batch)(tt, catT)
    return jnp.transpose(out, (1, 0))
